# baseline (speedup 1.0000x reference)
import functools

import jax
import jax.numpy as jnp
from jax import lax
from jax.experimental import pallas as pl
from jax.experimental.pallas import tpu as pltpu

N = 10000
E = 160000
D = 256
DE = 16
H = 256

NODE_BLK = 1000
EDGE_BLK = 1600


def _node_pre_body(nf_ref, ws_ref, bs_ref, wd_ref, bd_ref, s_ref, d_ref):
    nf = nf_ref[...]
    s_ref[...] = jnp.dot(nf, ws_ref[...], preferred_element_type=jnp.float32) + bs_ref[...]
    d_ref[...] = jnp.dot(nf, wd_ref[...], preferred_element_type=jnp.float32) + bd_ref[...]


def _node_precompute(node_feat, Ws, bs, Wd, bd):
    grid = (N // NODE_BLK,)
    return pl.pallas_call(
        _node_pre_body,
        grid=grid,
        in_specs=[
            pl.BlockSpec((NODE_BLK, D), lambda i: (i, 0)),
            pl.BlockSpec((D, H), lambda i: (0, 0)),
            pl.BlockSpec((H,), lambda i: (0,)),
            pl.BlockSpec((D, H), lambda i: (0, 0)),
            pl.BlockSpec((H,), lambda i: (0,)),
        ],
        out_specs=[
            pl.BlockSpec((NODE_BLK, H), lambda i: (i, 0)),
            pl.BlockSpec((NODE_BLK, H), lambda i: (i, 0)),
        ],
        out_shape=[
            jax.ShapeDtypeStruct((N, H), jnp.float32),
            jax.ShapeDtypeStruct((N, H), jnp.float32),
        ],
    )(node_feat, Ws, bs, Wd, bd)


def _edge_mlp_body(ea_ref, gsd_ref, we1_ref, be1_ref, we2_ref, be2_ref,
                   wt1_ref, bt1_ref, wt2_ref, bt2_ref, m_ref):
    h1 = jnp.maximum(jnp.dot(ea_ref[...], we1_ref[...],
                             preferred_element_type=jnp.float32) + be1_ref[...], 0.0)
    ec = jnp.dot(h1, we2_ref[...], preferred_element_type=jnp.float32) + be2_ref[...]
    m1 = jnp.maximum(ec + gsd_ref[...], 0.0)
    m2 = jnp.maximum(jnp.dot(m1, wt1_ref[...],
                             preferred_element_type=jnp.float32) + bt1_ref[...], 0.0)
    m_ref[...] = jnp.dot(m2, wt2_ref[...], preferred_element_type=jnp.float32) + bt2_ref[...]


def _edge_mlp(edge_attr, gsd, We1, be1, We2, be2, Wt1, bt1, Wt2, bt2):
    grid = (E // EDGE_BLK,)
    return pl.pallas_call(
        _edge_mlp_body,
        grid=grid,
        in_specs=[
            pl.BlockSpec((EDGE_BLK, DE), lambda i: (i, 0)),
            pl.BlockSpec((EDGE_BLK, H), lambda i: (i, 0)),
            pl.BlockSpec((DE, H), lambda i: (0, 0)),
            pl.BlockSpec((H,), lambda i: (0,)),
            pl.BlockSpec((H, H), lambda i: (0, 0)),
            pl.BlockSpec((H,), lambda i: (0,)),
            pl.BlockSpec((H, H), lambda i: (0, 0)),
            pl.BlockSpec((H,), lambda i: (0,)),
            pl.BlockSpec((H, D), lambda i: (0, 0)),
            pl.BlockSpec((D,), lambda i: (0,)),
        ],
        out_specs=pl.BlockSpec((EDGE_BLK, D), lambda i: (i, 0)),
        out_shape=jax.ShapeDtypeStruct((E, D), jnp.float32),
    )(edge_attr, gsd, We1, be1, We2, be2, Wt1, bt1, Wt2, bt2)


def _final_body(nf_ref, agg_ref, wpd_ref, bpd_ref, wpe_ref, bpe_ref,
                wp_ref, bp_ref, out_ref):
    z = (jnp.dot(nf_ref[...], wpd_ref[...], preferred_element_type=jnp.float32)
         + bpd_ref[...]
         + jnp.dot(agg_ref[...], wpe_ref[...], preferred_element_type=jnp.float32)
         + bpe_ref[...])
    out_ref[...] = jnp.dot(jnp.maximum(z, 0.0), wp_ref[...],
                           preferred_element_type=jnp.float32) + bp_ref[...]


def _final_stage(node_feat, agg, Wpd, bpd, Wpe, bpe, Wp, bp):
    grid = (N // NODE_BLK,)
    return pl.pallas_call(
        _final_body,
        grid=grid,
        in_specs=[
            pl.BlockSpec((NODE_BLK, D), lambda i: (i, 0)),
            pl.BlockSpec((NODE_BLK, D), lambda i: (i, 0)),
            pl.BlockSpec((D, H), lambda i: (0, 0)),
            pl.BlockSpec((H,), lambda i: (0,)),
            pl.BlockSpec((D, H), lambda i: (0, 0)),
            pl.BlockSpec((H,), lambda i: (0,)),
            pl.BlockSpec((H, D), lambda i: (0, 0)),
            pl.BlockSpec((D,), lambda i: (0,)),
        ],
        out_specs=pl.BlockSpec((NODE_BLK, D), lambda i: (i, 0)),
        out_shape=jax.ShapeDtypeStruct((N, D), jnp.float32),
    )(node_feat, agg, Wpd, bpd, Wpe, bpe, Wp, bp)


def kernel(node_feat, edge_index, edge_attr, We1, be1, We2, be2, Ws, bs, Wd, bd,
           Wt1, bt1, Wt2, bt2, Wpd, bpd, Wpe, bpe, Wp, bp):
    src = edge_index[0]
    dst = edge_index[1]
    s, d = _node_precompute(node_feat, Ws, bs, Wd, bd)
    gsd = s[src] + d[dst]
    m = _edge_mlp(edge_attr, gsd, We1, be1, We2, be2, Wt1, bt1, Wt2, bt2)
    agg = jax.ops.segment_sum(m, dst, num_segments=N)
    return _final_stage(node_feat, agg, Wpd, bpd, Wpe, bpe, Wp, bp)


# speedup vs baseline: 2.1352x; 2.1352x over previous
import functools

import jax
import jax.numpy as jnp
from jax import lax
from jax.experimental import pallas as pl
from jax.experimental.pallas import tpu as pltpu
from jax.experimental.pallas import tpu_sc as plsc

N = 10000
E = 160000
D = 256
DE = 16
H = 256

NODE_BLK = 1000
EDGE_BLK = 1600

NC = 2
NS = 16
NW = NC * NS
GB = 200
SB = 200
HH = H // 2
NSTRIPE = N // NS


def _node_pre_body(nf_ref, ws_ref, bs_ref, wd_ref, bd_ref, s_ref, d_ref):
    nf = nf_ref[...]
    s_ref[...] = jnp.dot(nf, ws_ref[...], preferred_element_type=jnp.float32) + bs_ref[...]
    d_ref[...] = jnp.dot(nf, wd_ref[...], preferred_element_type=jnp.float32) + bd_ref[...]


def _node_precompute(node_feat, Ws, bs, Wd, bd):
    return pl.pallas_call(
        _node_pre_body,
        grid=(N // NODE_BLK,),
        in_specs=[
            pl.BlockSpec((NODE_BLK, D), lambda i: (i, 0)),
            pl.BlockSpec((D, H), lambda i: (0, 0)),
            pl.BlockSpec((H,), lambda i: (0,)),
            pl.BlockSpec((D, H), lambda i: (0, 0)),
            pl.BlockSpec((H,), lambda i: (0,)),
        ],
        out_specs=[
            pl.BlockSpec((NODE_BLK, H), lambda i: (i, 0)),
            pl.BlockSpec((NODE_BLK, H), lambda i: (i, 0)),
        ],
        out_shape=[
            jax.ShapeDtypeStruct((N, H), jnp.float32),
            jax.ShapeDtypeStruct((N, H), jnp.float32),
        ],
    )(node_feat, Ws, bs, Wd, bd)


def _sc_gather(s, d, src, dst):
    per_w = E // NW
    n_it = per_w // GB

    mesh = plsc.VectorSubcoreMesh(core_axis_name="c", subcore_axis_name="s")

    @functools.partial(
        pl.kernel,
        mesh=mesh,
        out_type=[
            jax.ShapeDtypeStruct((E, H), jnp.float32),
            jax.ShapeDtypeStruct((E, H), jnp.float32),
        ],
        scratch_types=[
            pltpu.VMEM((GB,), jnp.int32),
            pltpu.VMEM((GB,), jnp.int32),
            pltpu.VMEM((GB, H), jnp.float32),
            pltpu.VMEM((GB, H), jnp.float32),
            pltpu.SemaphoreType.DMA,
            pltpu.SemaphoreType.DMA,
        ],
    )
    def k(s_hbm, d_hbm, src_hbm, dst_hbm, gs_hbm, gd_hbm,
          idx_s, idx_d, rows_s, rows_d, sem_s, sem_d):
        wid = lax.axis_index("s") * NC + lax.axis_index("c")
        base = wid * per_w

        def body(i, carry):
            off = base + i * GB
            pltpu.sync_copy(src_hbm.at[pl.ds(off, GB)], idx_s)
            pltpu.sync_copy(dst_hbm.at[pl.ds(off, GB)], idx_d)
            cp_s = pltpu.async_copy(s_hbm.at[idx_s], rows_s, sem_s)
            cp_d = pltpu.async_copy(d_hbm.at[idx_d], rows_d, sem_d)
            cp_s.wait()
            cp_d.wait()
            pltpu.sync_copy(rows_s, gs_hbm.at[pl.ds(off, GB)])
            pltpu.sync_copy(rows_d, gd_hbm.at[pl.ds(off, GB)])
            return carry

        lax.fori_loop(0, n_it, body, 0)

    return k(s, d, src, dst)


def _edge_mlp_body(ea_ref, gs_ref, gd_ref, we1_ref, be1_ref, we2_ref, be2_ref,
                   wt1_ref, bt1_ref, wt2_ref, bt2_ref, m_ref):
    h1 = jnp.maximum(jnp.dot(ea_ref[...], we1_ref[...],
                             preferred_element_type=jnp.float32) + be1_ref[...], 0.0)
    ec = jnp.dot(h1, we2_ref[...], preferred_element_type=jnp.float32) + be2_ref[...]
    m1 = jnp.maximum(ec + gs_ref[...] + gd_ref[...], 0.0)
    m2 = jnp.maximum(jnp.dot(m1, wt1_ref[...],
                             preferred_element_type=jnp.float32) + bt1_ref[...], 0.0)
    m_ref[...] = jnp.dot(m2, wt2_ref[...], preferred_element_type=jnp.float32) + bt2_ref[...]


def _edge_mlp(edge_attr, gs, gd, We1, be1, We2, be2, Wt1, bt1, Wt2, bt2):
    return pl.pallas_call(
        _edge_mlp_body,
        grid=(E // EDGE_BLK,),
        in_specs=[
            pl.BlockSpec((EDGE_BLK, DE), lambda i: (i, 0)),
            pl.BlockSpec((EDGE_BLK, H), lambda i: (i, 0)),
            pl.BlockSpec((EDGE_BLK, H), lambda i: (i, 0)),
            pl.BlockSpec((DE, H), lambda i: (0, 0)),
            pl.BlockSpec((H,), lambda i: (0,)),
            pl.BlockSpec((H, H), lambda i: (0, 0)),
            pl.BlockSpec((H,), lambda i: (0,)),
            pl.BlockSpec((H, H), lambda i: (0, 0)),
            pl.BlockSpec((H,), lambda i: (0,)),
            pl.BlockSpec((H, D), lambda i: (0, 0)),
            pl.BlockSpec((D,), lambda i: (0,)),
        ],
        out_specs=pl.BlockSpec((EDGE_BLK, D), lambda i: (i, 0)),
        out_shape=jax.ShapeDtypeStruct((E, D), jnp.float32),
    )(edge_attr, gs, gd, We1, be1, We2, be2, Wt1, bt1, Wt2, bt2)


def _sc_scatter_add(m3, dst, zeros_stripe):
    per_t = E // NS
    n_it = per_t // SB

    mesh = plsc.VectorSubcoreMesh(core_axis_name="c", subcore_axis_name="s")

    @functools.partial(
        pl.kernel,
        mesh=mesh,
        out_type=jax.ShapeDtypeStruct((N, NC, HH), jnp.float32),
        scratch_types=[
            pltpu.VMEM((SB,), jnp.int32),
            pltpu.VMEM((SB, HH), jnp.float32),
            pltpu.VMEM_SHARED((N, HH), jnp.float32),
        ],
    )
    def k(m_hbm, dst_hbm, z_hbm, out_hbm, idx_v, blk_v, acc):
        c = lax.axis_index("c")
        t = lax.axis_index("s")

        pltpu.sync_copy(z_hbm, acc.at[pl.ds(t * NSTRIPE, NSTRIPE)])
        plsc.subcore_barrier()

        def body(i, carry):
            off = t * per_t + i * SB
            pltpu.sync_copy(dst_hbm.at[pl.ds(off, SB)], idx_v)
            pltpu.sync_copy(m_hbm.at[pl.ds(off, SB), c], blk_v)
            pltpu.sync_copy(blk_v, acc.at[idx_v], add=True)
            return carry

        lax.fori_loop(0, n_it, body, 0)
        plsc.subcore_barrier()

        pltpu.sync_copy(acc.at[pl.ds(t * NSTRIPE, NSTRIPE)],
                        out_hbm.at[pl.ds(t * NSTRIPE, NSTRIPE), c])

    return k(m3, dst, zeros_stripe)


def _final_body(nf_ref, agg_ref, wpd_ref, bpd_ref, wpe_ref, bpe_ref,
                wp_ref, bp_ref, out_ref):
    z = (jnp.dot(nf_ref[...], wpd_ref[...], preferred_element_type=jnp.float32)
         + bpd_ref[...]
         + jnp.dot(agg_ref[...], wpe_ref[...], preferred_element_type=jnp.float32)
         + bpe_ref[...])
    out_ref[...] = jnp.dot(jnp.maximum(z, 0.0), wp_ref[...],
                           preferred_element_type=jnp.float32) + bp_ref[...]


def _final_stage(node_feat, agg, Wpd, bpd, Wpe, bpe, Wp, bp):
    return pl.pallas_call(
        _final_body,
        grid=(N // NODE_BLK,),
        in_specs=[
            pl.BlockSpec((NODE_BLK, D), lambda i: (i, 0)),
            pl.BlockSpec((NODE_BLK, D), lambda i: (i, 0)),
            pl.BlockSpec((D, H), lambda i: (0, 0)),
            pl.BlockSpec((H,), lambda i: (0,)),
            pl.BlockSpec((D, H), lambda i: (0, 0)),
            pl.BlockSpec((H,), lambda i: (0,)),
            pl.BlockSpec((H, D), lambda i: (0, 0)),
            pl.BlockSpec((D,), lambda i: (0,)),
        ],
        out_specs=pl.BlockSpec((NODE_BLK, D), lambda i: (i, 0)),
        out_shape=jax.ShapeDtypeStruct((N, D), jnp.float32),
    )(node_feat, agg, Wpd, bpd, Wpe, bpe, Wp, bp)


def kernel(node_feat, edge_index, edge_attr, We1, be1, We2, be2, Ws, bs, Wd, bd,
           Wt1, bt1, Wt2, bt2, Wpd, bpd, Wpe, bpe, Wp, bp):
    src = edge_index[0]
    dst = edge_index[1]
    s, d = _node_precompute(node_feat, Ws, bs, Wd, bd)
    gs, gd = _sc_gather(s, d, src, dst)
    m = _edge_mlp(edge_attr, gs, gd, We1, be1, We2, be2, Wt1, bt1, Wt2, bt2)
    zeros_stripe = jnp.zeros((NSTRIPE, HH), jnp.float32)
    agg3 = _sc_scatter_add(m.reshape(E, NC, HH), dst, zeros_stripe)
    agg = agg3.reshape(N, D)
    return _final_stage(node_feat, agg, Wpd, bpd, Wpe, bpe, Wp, bp)
